# Initial kernel scaffold; baseline (speedup 1.0000x reference)
import jax
import jax.numpy as jnp
from jax.experimental import pallas as pl


def kernel(digit_bundles, ns):
    raise NotImplementedError("write your pallas kernel here")


# speedup vs baseline: 3.5530x; 3.5530x over previous
import jax
import jax.numpy as jnp
from jax import lax
from jax.experimental import pallas as pl
from jax.experimental.pallas import tpu as pltpu
from jax.experimental.pallas import tpu_sc as plsc

MAX_POS = 8
PER_DIGIT_DIM = 64
BATCH = 16384

_NPAIR = MAX_POS // 2
_PAIR_DIM = 2 * PER_DIGIT_DIM
_NC = 2
_NS = 16
_NW = _NC * _NS
_B_PER_W = BATCH // _NW
_ROWS_PER_W = _B_PER_W * _NPAIR
_CHUNK = 512


def _sc_body(pair_hbm, ns_hbm, out_hbm, ns_v, idx_v, rows_v, sem):
    wid = lax.axis_index("s") * _NC + lax.axis_index("c")
    base_b = wid * _B_PER_W

    pltpu.sync_copy(ns_hbm.at[pl.ds(base_b, _B_PER_W)], ns_v)

    lane4 = lax.iota(jnp.int32, 16) * _NPAIR

    def build(g, carry):
        q = ns_v[pl.ds(g * 16, 16)]
        base_idx = lane4 + g * (16 * _NPAIR)
        for i in range(_NPAIR):
            q2 = q // 100
            p = q - q2 * 100
            plsc.store_scatter(idx_v, [base_idx + i], p)
            q = q2
        return carry

    lax.fori_loop(0, _B_PER_W // 16, build, 0)

    for c in range(_ROWS_PER_W // _CHUNK):
        pltpu.async_copy(
            pair_hbm.at[idx_v.at[pl.ds(c * _CHUNK, _CHUNK)]], rows_v, sem
        ).wait()
        pltpu.sync_copy(
            rows_v, out_hbm.at[pl.ds(wid * _ROWS_PER_W + c * _CHUNK, _CHUNK)]
        )


@jax.jit
def _run(digit_bundles, ns):
    lo = jnp.broadcast_to(digit_bundles[None, :, :], (10, 10, PER_DIGIT_DIM))
    hi = jnp.broadcast_to(digit_bundles[:, None, :], (10, 10, PER_DIGIT_DIM))
    pair = jnp.concatenate(
        [lo.reshape(100, PER_DIGIT_DIM), hi.reshape(100, PER_DIGIT_DIM)], axis=-1
    )

    mesh = plsc.VectorSubcoreMesh(core_axis_name="c", subcore_axis_name="s")
    call = pl.kernel(
        _sc_body,
        out_type=jax.ShapeDtypeStruct((BATCH * _NPAIR, _PAIR_DIM), jnp.float32),
        mesh=mesh,
        scratch_types=[
            pltpu.VMEM((_B_PER_W,), jnp.int32),
            pltpu.VMEM((_ROWS_PER_W,), jnp.int32),
            pltpu.VMEM((_CHUNK, _PAIR_DIM), jnp.float32),
            pltpu.SemaphoreType.DMA,
        ],
        compiler_params=pltpu.CompilerParams(needs_layout_passes=False),
    )
    rows = call(pair, ns)
    return rows.reshape(BATCH, MAX_POS * PER_DIGIT_DIM)


def kernel(digit_bundles, ns):
    return _run(digit_bundles, ns.astype(jnp.int32))
